# baseline (speedup 1.0000x reference)
import functools

import jax
import jax.numpy as jnp
from jax import lax
from jax.experimental import pallas as pl
from jax.experimental.pallas import tpu as pltpu
from jax.experimental.pallas import tpu_sc as plsc

DIM = 32
BATCH = 16384
NROWS = 1000000
L = 16
NC = 2
NS = 16
NW = NC * NS
BPW = BATCH // NW
CHUNK = 128
NCH = BPW // CHUNK

TCW = 512
TC_GRID = -(-NROWS // TCW)
NLINES = TC_GRID * CHUNK


def _pack_body(e_ref, r_ref, oe_ref, or_ref):
    for x_ref, o_ref in ((e_ref, oe_ref), (r_ref, or_ref)):
        x = x_ref[...]
        n = jnp.sum(jnp.abs(x), axis=0, keepdims=True)
        y = x / n
        rows = []
        for u in range(4):
            rows.append(jnp.concatenate(
                [jnp.transpose(y[:, u * 128 + a * 32:u * 128 + (a + 1) * 32])
                 for a in range(4)], axis=1))
        o_ref[...] = jnp.concatenate(rows, axis=0)


_packT = pl.pallas_call(
    _pack_body,
    grid=(TC_GRID,),
    in_specs=[
        pl.BlockSpec((DIM, TCW), lambda s: (0, s)),
        pl.BlockSpec((DIM, TCW), lambda s: (0, s)),
    ],
    out_specs=[
        pl.BlockSpec((CHUNK, 128), lambda s: (s, 0)),
        pl.BlockSpec((CHUNK, 128), lambda s: (s, 0)),
    ],
    out_shape=[
        jax.ShapeDtypeStruct((NLINES, 128), jnp.float32),
        jax.ShapeDtypeStruct((NLINES, 128), jnp.float32),
    ],
)

_mesh = plsc.VectorSubcoreMesh(core_axis_name="c", subcore_axis_name="s")


@functools.partial(
    pl.kernel,
    out_type=jax.ShapeDtypeStruct((BATCH,), jnp.float32),
    mesh=_mesh,
    scratch_types=[
        pltpu.VMEM((3 * BPW,), jnp.int32),
        pltpu.VMEM((3 * BPW,), jnp.int32),
        pltpu.VMEM((3 * BPW,), jnp.int32),
        pltpu.VMEM((3, CHUNK, 128), jnp.float32),
        pltpu.VMEM((3, CHUNK, 128), jnp.float32),
        pltpu.VMEM((BPW,), jnp.float32),
        pltpu.SemaphoreType.DMA,
        pltpu.SemaphoreType.DMA,
    ],
    compiler_params=pltpu.CompilerParams(needs_layout_passes=False),
)
def _transe_sc(ent_l, rel_l, idx, out,
               idx_v, line_v, offs_v, st0, st1, out_v, sem0, sem1):
    wid = lax.axis_index("s") * NC + lax.axis_index("c")
    base = wid * BPW

    pltpu.sync_copy(idx.at[pl.ds(wid * (3 * BPW), 3 * BPW)], idx_v)

    def mkline(i, carry):
        v = idx_v[pl.ds(i * L, L)]
        line_v[pl.ds(i * L, L)] = (
            lax.shift_right_logical(v, 7) * 32 + (v & 31))
        offs_v[pl.ds(i * L, L)] = (
            (lax.shift_right_logical(v, 5) & 3) * DIM)
        return carry

    lax.fori_loop(0, (3 * BPW) // L, mkline, 0)

    tables = (ent_l, rel_l, ent_l)
    stages = (st0, st1)
    sems = (sem0, sem1)

    def fire(k):
        st = stages[k % 2]
        sem = sems[k % 2]
        return [
            pltpu.async_copy(
                tables[t].at[line_v.at[pl.ds(t * BPW + k * CHUNK, CHUNK)]],
                st.at[t], sem)
            for t in range(3)
        ]

    pending = {0: fire(0)}
    for k in range(NCH):
        if k + 1 < NCH:
            pending[k + 1] = fire(k + 1)
        for c in pending.pop(k):
            c.wait()
        st = stages[k % 2]

        def group(g, carry, k=k, st=st):
            lanes = g * L + lax.iota(jnp.int32, L)
            tsel = [jnp.full((L,), t, jnp.int32) for t in range(3)]
            offs = []
            for t in range(3):
                offs.append(offs_v[pl.ds(t * BPW + k * CHUNK + g * L, L)])
            acc = jnp.zeros((L,), jnp.float32)
            for j in range(DIM):
                a = plsc.load_gather(st, [tsel[0], lanes, offs[0] + j])
                b = plsc.load_gather(st, [tsel[1], lanes, offs[1] + j])
                d = plsc.load_gather(st, [tsel[2], lanes, offs[2] + j])
                acc = acc + jnp.abs(a + b - d)
            out_v[pl.ds(k * CHUNK + g * L, L)] = acc
            return carry

        lax.fori_loop(0, CHUNK // L, group, 0)

    pltpu.sync_copy(out_v, out.at[pl.ds(base, BPW)])


@jax.jit
def kernel(batch_inputs, entity_weight, relation_weight):
    bi = batch_inputs.astype(jnp.int32)
    idx = bi.reshape(NW, BPW, 3).transpose(0, 2, 1).reshape(NW * 3 * BPW)
    ent_l, rel_l = _packT(entity_weight.T, relation_weight.T)
    return _transe_sc(ent_l, rel_l, idx)


# speedup vs baseline: 2.8302x; 2.8302x over previous
import functools

import jax
import jax.numpy as jnp
from jax import lax
from jax.experimental import pallas as pl
from jax.experimental.pallas import tpu as pltpu
from jax.experimental.pallas import tpu_sc as plsc

DIM = 32
BATCH = 16384
L = 16
NC = 2
NS = 16
NW = NC * NS
BPW = BATCH // NW
CHUNK = 128
NCH = BPW // CHUNK

_mesh = plsc.VectorSubcoreMesh(core_axis_name="c", subcore_axis_name="s")


@functools.partial(
    pl.kernel,
    out_type=jax.ShapeDtypeStruct((BATCH,), jnp.float32),
    mesh=_mesh,
    scratch_types=[
        pltpu.VMEM((3, NCH, CHUNK), jnp.int32),
        pltpu.VMEM((BPW, DIM), jnp.float32),
        pltpu.VMEM((BPW, DIM), jnp.float32),
        pltpu.VMEM((BPW, DIM), jnp.float32),
        pltpu.VMEM((BPW,), jnp.float32),
        pltpu.SemaphoreType.DMA,
    ],
    compiler_params=pltpu.CompilerParams(
        needs_layout_passes=False, use_tc_tiling_on_sc=False),
)
def _transe_sc(ent, rel, idx, out, idx_v, r1_v, rr_v, r2_v, out_v, sem):
    wid = lax.axis_index("s") * NC + lax.axis_index("c")
    base = wid * BPW

    pltpu.sync_copy(idx.at[wid], idx_v)

    copies = []
    for k in range(NCH):
        dst = pl.ds(k * CHUNK, CHUNK)
        copies.append(pltpu.async_copy(ent.at[idx_v.at[0, k]], r1_v.at[dst], sem))
        copies.append(pltpu.async_copy(rel.at[idx_v.at[1, k]], rr_v.at[dst], sem))
        copies.append(pltpu.async_copy(ent.at[idx_v.at[2, k]], r2_v.at[dst], sem))
    for c in copies:
        c.wait()

    cols = [jnp.full((L,), j, jnp.int32) for j in range(DIM)]

    def group(g, carry):
        rows = g * L + lax.iota(jnp.int32, L)
        n1 = jnp.zeros((L,), jnp.float32)
        nr = jnp.zeros((L,), jnp.float32)
        n2 = jnp.zeros((L,), jnp.float32)
        for j in range(DIM):
            n1 = n1 + jnp.abs(plsc.load_gather(r1_v, [rows, cols[j]]))
            nr = nr + jnp.abs(plsc.load_gather(rr_v, [rows, cols[j]]))
            n2 = n2 + jnp.abs(plsc.load_gather(r2_v, [rows, cols[j]]))
        s1 = 1.0 / n1
        sr = 1.0 / nr
        s2 = 1.0 / n2
        acc = jnp.zeros((L,), jnp.float32)
        for j in range(DIM):
            a = plsc.load_gather(r1_v, [rows, cols[j]])
            b = plsc.load_gather(rr_v, [rows, cols[j]])
            d = plsc.load_gather(r2_v, [rows, cols[j]])
            acc = acc + jnp.abs(a * s1 + b * sr - d * s2)
        out_v[pl.ds(g * L, L)] = acc
        return carry

    lax.fori_loop(0, BPW // L, group, 0)

    pltpu.sync_copy(out_v, out.at[pl.ds(base, BPW)])


@jax.jit
def kernel(batch_inputs, entity_weight, relation_weight):
    bi = batch_inputs.astype(jnp.int32)
    idx = bi.reshape(NW, NCH, CHUNK, 3).transpose(0, 3, 1, 2)
    return _transe_sc(entity_weight, relation_weight, idx)
